# baseline (device time: 37816 ns/iter reference)
import jax
import jax.numpy as jnp
from jax import lax
from jax.experimental import pallas as pl
from jax.experimental.pallas import tpu as pltpu

N_DEV = 16
PLANE = 4
N_Z = 4
N_TOK = 1024
D_IN = 512
D_OUT = 1024
E_LOCAL = 4
ROWS = N_TOK // N_DEV
CAP1 = 16


def kernel(x, router_W, route_idx, expert_W, shared_W):
    def body(x_ref, rw_ref, ri_ref, ew_ref, sw_ref, out_ref,
             partial_bf_ref, stage_ref, rp1_ref, rp2_ref, scale_ref,
             p1_send_sems, p1_recv_sems, p2_send_sems, p2_recv_sems):
        d = lax.axis_index("i")
        my_z = lax.div(d, PLANE)
        my_w = lax.rem(d, PLANE)

        scores = x_ref[:, :] @ rw_ref[:, :]
        m = jnp.max(scores, axis=-1, keepdims=True)
        p = jnp.exp(scores - m)
        probs = p / jnp.sum(p, axis=-1, keepdims=True)
        route = ri_ref[:, :]
        eids = lax.broadcasted_iota(route.dtype, scores.shape, 1)
        coef = jnp.sum(jnp.where(eids == route, probs, 0.0),
                       axis=-1, keepdims=True)
        scale_ref[:, :] = jnp.concatenate(
            [jnp.where(route == d * E_LOCAL + e, coef, 0.0)
             for e in range(E_LOCAL)], axis=-1)

        barrier = pltpu.get_barrier_semaphore()
        for o in range(1, PLANE):
            wp = lax.rem(my_w + o, PLANE)
            pl.semaphore_signal(barrier, inc=1, device_id=(my_z * PLANE + wp,),
                                device_id_type=pl.DeviceIdType.MESH)
        for o in range(1, N_Z):
            zq = lax.rem(my_z + o, N_Z)
            pl.semaphore_signal(barrier, inc=1, device_id=(zq * PLANE + my_w,),
                                device_id_type=pl.DeviceIdType.MESH)
        pl.semaphore_wait(barrier, 6)

        sends = []

        tri = (lax.broadcasted_iota(jnp.int32, (ROWS, ROWS), 1)
               <= lax.broadcasted_iota(jnp.int32, (ROWS, ROWS), 0)
               ).astype(jnp.float32)
        iota_cap = lax.broadcasted_iota(jnp.int32, (ROWS, CAP1), 1).astype(jnp.float32)

        def perm_matrix(chunk, dev):
            route_c = ri_ref[pl.ds(chunk * ROWS, ROWS), :]
            mask = jnp.logical_and(route_c >= dev * E_LOCAL,
                                   route_c < (dev + 1) * E_LOCAL)
            maskf = mask.astype(jnp.float32)
            slots = tri @ maskf - 1.0
            return jnp.where(mask, (slots == iota_cap).astype(jnp.float32),
                             0.0)

        BLK = PLANE * ROWS
        for k in range(1, N_Z + 1):
            zk = lax.rem(my_z + k, N_Z)
            rs = zk * BLK
            xb = x_ref[pl.ds(rs, BLK), :]
            acc = (xb * scale_ref[pl.ds(rs, BLK), 0:1]) @ ew_ref[0]
            for e in range(1, E_LOCAL):
                acc += (xb * scale_ref[pl.ds(rs, BLK), e:e + 1]) @ ew_ref[e]
            partial_bf_ref[pl.ds(rs, BLK), :] = acc.astype(jnp.bfloat16)
            for o in range(1, PLANE):
                wp = lax.rem(my_w + o, PLANE)
                dest = my_z * PLANE + wp
                chunk = 4 * zk + wp
                pt = perm_matrix(chunk, d)
                chunk_f = partial_bf_ref[pl.ds(chunk * ROWS, ROWS), :].astype(jnp.float32)
                g = lax.dot_general(pt, chunk_f, (((0,), (0,)), ((), ())),
                                    preferred_element_type=jnp.float32)
                stage_ref[wp, zk] = g.astype(jnp.bfloat16)
                rdma = pltpu.make_async_remote_copy(
                    src_ref=stage_ref.at[wp, zk],
                    dst_ref=rp1_ref.at[my_w, zk],
                    send_sem=p1_send_sems.at[wp, zk],
                    recv_sem=p1_recv_sems.at[my_w, zk],
                    device_id=(dest,),
                    device_id_type=pl.DeviceIdType.MESH,
                )
                rdma.start()
                sends.append(rdma)

        def accum_zgroup(zk):
            chunk = 4 * zk + my_w
            acc2 = partial_bf_ref[pl.ds(chunk * ROWS, ROWS), :].astype(jnp.float32)
            for oo in range(1, PLANE):
                wq = lax.rem(my_w + oo, PLANE)
                recv = pltpu.make_async_remote_copy(
                    src_ref=stage_ref.at[0, 0],
                    dst_ref=rp1_ref.at[wq, zk],
                    send_sem=p1_send_sems.at[wq, zk],
                    recv_sem=p1_recv_sems.at[wq, zk],
                    device_id=(d,),
                    device_id_type=pl.DeviceIdType.MESH,
                )
                recv.wait_recv()
                pt = perm_matrix(chunk, my_z * PLANE + wq)
                acc2 = acc2 + pt @ rp1_ref[wq, zk].astype(jnp.float32)
            return acc2

        for o in range(1, N_Z):
            zq = lax.rem(my_z + o, N_Z)
            dest = zq * PLANE + my_w
            acc2 = accum_zgroup(zq)
            partial_bf_ref[pl.ds(dest * ROWS, ROWS), :] = acc2.astype(jnp.bfloat16)
            rdma = pltpu.make_async_remote_copy(
                src_ref=partial_bf_ref.at[pl.ds(dest * ROWS, ROWS), :],
                dst_ref=rp2_ref.at[my_z],
                send_sem=p2_send_sems.at[zq],
                recv_sem=p2_recv_sems.at[my_z],
                device_id=(dest,),
                device_id_type=pl.DeviceIdType.MESH,
            )
            rdma.start()
            sends.append(rdma)

        result = accum_zgroup(my_z)
        result += x_ref[pl.ds(d * ROWS, ROWS), :] @ sw_ref[:, :]

        for o in range(1, N_Z):
            zr = lax.rem(my_z + o, N_Z)
            recv = pltpu.make_async_remote_copy(
                src_ref=partial_bf_ref.at[pl.ds(0, ROWS), :],
                dst_ref=rp2_ref.at[zr],
                send_sem=p2_send_sems.at[zr],
                recv_sem=p2_recv_sems.at[zr],
                device_id=(d,),
                device_id_type=pl.DeviceIdType.MESH,
            )
            recv.wait_recv()
            result += rp2_ref[zr].astype(jnp.float32)
        out_ref[:, :] = result

        for rdma in sends:
            rdma.wait_send()

    return pl.pallas_call(
        body,
        out_shape=jax.ShapeDtypeStruct((ROWS, D_OUT), jnp.float32),
        in_specs=[pl.BlockSpec(memory_space=pltpu.VMEM)] * 5,
        out_specs=pl.BlockSpec(memory_space=pltpu.VMEM),
        scratch_shapes=[
            pltpu.VMEM((N_TOK, D_OUT), jnp.bfloat16),
            pltpu.VMEM((PLANE, N_Z, CAP1, D_OUT), jnp.bfloat16),
            pltpu.VMEM((PLANE, N_Z, CAP1, D_OUT), jnp.bfloat16),
            pltpu.VMEM((N_Z, ROWS, D_OUT), jnp.bfloat16),
            pltpu.VMEM((N_TOK, E_LOCAL), jnp.float32),
            pltpu.SemaphoreType.DMA((PLANE, N_Z)),
            pltpu.SemaphoreType.DMA((PLANE, N_Z)),
            pltpu.SemaphoreType.DMA((N_Z,)),
            pltpu.SemaphoreType.DMA((N_Z,)),
        ],
        compiler_params=pltpu.CompilerParams(collective_id=0),
    )(x, router_W, route_idx, expert_W, shared_W)


# device time: 31262 ns/iter; 1.2096x vs baseline; 1.2096x over previous
import jax
import jax.numpy as jnp
from jax import lax
from jax.experimental import pallas as pl
from jax.experimental.pallas import tpu as pltpu

N_DEV = 16
PLANE = 4
N_Z = 4
N_TOK = 1024
D_IN = 512
D_OUT = 1024
E_LOCAL = 4
ROWS = N_TOK // N_DEV
BLK = PLANE * ROWS
CAP1 = 16


def kernel(x, router_W, route_idx, expert_W, shared_W):
    def body(x_ref, rw_ref, ri_ref, ew_ref, sw_ref, out_ref,
             stage_ref, rp_ref, scale_ref, send_sems, recv_sems):
        d = lax.axis_index("i")
        my_z = lax.div(d, PLANE)

        scores = x_ref[:, :] @ rw_ref[:, :]
        m = jnp.max(scores, axis=-1, keepdims=True)
        p = jnp.exp(scores - m)
        probs = p / jnp.sum(p, axis=-1, keepdims=True)
        route = ri_ref[:, :]
        eids = lax.broadcasted_iota(route.dtype, scores.shape, 1)
        coef = jnp.sum(jnp.where(eids == route, probs, 0.0),
                       axis=-1, keepdims=True)
        scale_ref[:, :] = jnp.concatenate(
            [jnp.where(route == d * E_LOCAL + e, coef, 0.0)
             for e in range(E_LOCAL)], axis=-1)

        barrier = pltpu.get_barrier_semaphore()
        for j in range(N_DEV):
            pl.semaphore_signal(barrier, inc=1, device_id=(j,),
                                device_id_type=pl.DeviceIdType.MESH)
        pl.semaphore_wait(barrier, N_DEV)

        ri_b = lax.broadcasted_iota(jnp.int32, (BLK, BLK), 0)
        ci_b = lax.broadcasted_iota(jnp.int32, (BLK, BLK), 1)
        tri_bd = jnp.logical_and(ci_b <= ri_b,
                                 lax.div(ci_b, ROWS) == lax.div(ri_b, ROWS)
                                 ).astype(jnp.float32)
        iota_col4 = lax.broadcasted_iota(jnp.int32, (BLK, PLANE * CAP1), 1)

        sends = []

        for k in range(1, N_Z + 1):
            zk = lax.rem(my_z + k, N_Z)
            rs = zk * BLK
            xb = x_ref[pl.ds(rs, BLK), :]
            acc = (xb * scale_ref[pl.ds(rs, BLK), 0:1]) @ ew_ref[0]
            for e in range(1, E_LOCAL):
                acc += (xb * scale_ref[pl.ds(rs, BLK), e:e + 1]) @ ew_ref[e]

            route_b = ri_ref[pl.ds(rs, BLK), :]
            mine = lax.div(route_b, E_LOCAL) == d
            slots = tri_bd @ mine.astype(jnp.float32) - 1.0
            slots_i = slots.astype(jnp.int32)
            colidx = (lax.broadcasted_iota(jnp.int32, (BLK, 1), 0)
                      // ROWS) * CAP1 + slots_i
            ptb = jnp.where(jnp.logical_and(mine, slots_i < CAP1),
                            (colidx == iota_col4).astype(jnp.float32),
                            0.0)
            g = lax.dot_general(ptb, acc, (((0,), (0,)), ((), ())),
                                preferred_element_type=jnp.float32)
            stage_ref[pl.ds(zk * PLANE * CAP1, PLANE * CAP1), :] = (
                g.astype(jnp.bfloat16))

            for wp in range(PLANE):
                c = 4 * zk + wp
                rdma = pltpu.make_async_remote_copy(
                    src_ref=stage_ref.at[pl.ds(c * CAP1, CAP1), :],
                    dst_ref=rp_ref.at[pl.ds(d * CAP1, CAP1), :],
                    send_sem=send_sems.at[c],
                    recv_sem=recv_sems.at[d],
                    device_id=(c,),
                    device_id_type=pl.DeviceIdType.MESH,
                )
                rdma.start()
                sends.append(rdma)

        route_m = ri_ref[pl.ds(d * ROWS, ROWS), :]
        own_m = lax.div(route_m, E_LOCAL)
        onehot = (own_m == lax.broadcasted_iota(jnp.int32, (ROWS, N_DEV), 1)
                  ).astype(jnp.float32)
        tri64 = (lax.broadcasted_iota(jnp.int32, (ROWS, ROWS), 1)
                 <= lax.broadcasted_iota(jnp.int32, (ROWS, ROWS), 0)
                 ).astype(jnp.float32)
        counts = tri64 @ onehot
        slot_m = (jnp.sum(onehot * counts, axis=-1, keepdims=True)
                  ).astype(jnp.int32) - 1
        colidx_m = own_m * CAP1 + slot_m
        ptc = jnp.where(
            slot_m < CAP1,
            (colidx_m == lax.broadcasted_iota(
                jnp.int32, (ROWS, N_DEV * CAP1), 1)).astype(jnp.float32),
            0.0)

        shared = x_ref[pl.ds(d * ROWS, ROWS), :] @ sw_ref[:, :]

        for j in range(N_DEV):
            recv = pltpu.make_async_remote_copy(
                src_ref=stage_ref.at[pl.ds(0, CAP1), :],
                dst_ref=rp_ref.at[pl.ds(j * CAP1, CAP1), :],
                send_sem=send_sems.at[j],
                recv_sem=recv_sems.at[j],
                device_id=(d,),
                device_id_type=pl.DeviceIdType.MESH,
            )
            recv.wait_recv()
        out_ref[:, :] = shared + ptc @ rp_ref[:, :].astype(jnp.float32)

        for rdma in sends:
            rdma.wait_send()

    return pl.pallas_call(
        body,
        out_shape=jax.ShapeDtypeStruct((ROWS, D_OUT), jnp.float32),
        in_specs=[pl.BlockSpec(memory_space=pltpu.VMEM)] * 5,
        out_specs=pl.BlockSpec(memory_space=pltpu.VMEM),
        scratch_shapes=[
            pltpu.VMEM((N_DEV * CAP1, D_OUT), jnp.bfloat16),
            pltpu.VMEM((N_DEV * CAP1, D_OUT), jnp.bfloat16),
            pltpu.VMEM((N_TOK, E_LOCAL), jnp.float32),
            pltpu.SemaphoreType.DMA((N_DEV,)),
            pltpu.SemaphoreType.DMA((N_DEV,)),
        ],
        compiler_params=pltpu.CompilerParams(collective_id=0),
    )(x, router_W, route_idx, expert_W, shared_W)


# device time: 25516 ns/iter; 1.4821x vs baseline; 1.2252x over previous
import jax
import jax.numpy as jnp
from jax import lax
from jax.experimental import pallas as pl
from jax.experimental.pallas import tpu as pltpu

N_DEV = 16
PLANE = 4
N_Z = 4
N_TOK = 1024
D_IN = 512
D_OUT = 1024
E_LOCAL = 4
ROWS = N_TOK // N_DEV
BLK = PLANE * ROWS
CAP1 = 16
CAP_ME = 128


def kernel(x, router_W, route_idx, expert_W, shared_W):
    def body(x_ref, rw_ref, ri_ref, ew_ref, sw_ref, out_ref,
             stage_ref, rp_ref, send_sems, recv_sems):
        d = lax.axis_index("i")

        xv = x_ref[:, :]
        scores = xv @ rw_ref[:, :]
        m = jnp.max(scores, axis=-1, keepdims=True)
        p = jnp.exp(scores - m)
        probs = p / jnp.sum(p, axis=-1, keepdims=True)
        route = ri_ref[:, :]
        eids = lax.broadcasted_iota(route.dtype, scores.shape, 1)
        coef = jnp.sum(jnp.where(eids == route, probs, 0.0),
                       axis=-1, keepdims=True)
        scale_mat = jnp.concatenate(
            [jnp.where(route == d * E_LOCAL + e, coef, 0.0)
             for e in range(E_LOCAL)], axis=-1)

        barrier = pltpu.get_barrier_semaphore()
        for j in range(N_DEV):
            pl.semaphore_signal(barrier, inc=1, device_id=(j,),
                                device_id_type=pl.DeviceIdType.MESH)
        pl.semaphore_wait(barrier, N_DEV)

        ri_b = lax.broadcasted_iota(jnp.int32, (BLK, BLK), 0)
        ci_b = lax.broadcasted_iota(jnp.int32, (BLK, BLK), 1)
        tri_bd = jnp.logical_and(ci_b <= ri_b,
                                 lax.div(ci_b, ROWS) == lax.div(ri_b, ROWS)
                                 ).astype(jnp.float32)
        tri_lo = (ci_b <= ri_b).astype(jnp.float32)
        iota_col4 = lax.broadcasted_iota(jnp.int32, (BLK, PLANE * CAP1), 1)

        sends = []

        mine_all = lax.div(route, E_LOCAL) == d
        minef = mine_all.astype(jnp.float32)
        colg_blocks = []
        off = jnp.float32(0.0)
        for b in range(N_Z):
            mb = minef[b * BLK:(b + 1) * BLK]
            slots_b = tri_lo @ mb - 1.0 + off
            colg_blocks.append(slots_b)
            off = off + jnp.sum(mb)
        colg = jnp.concatenate(colg_blocks, axis=0).astype(jnp.int32)
        pg = jnp.where(
            jnp.logical_and(mine_all, colg < CAP_ME),
            (colg == lax.broadcasted_iota(jnp.int32, (N_TOK, CAP_ME), 1)
             ).astype(jnp.float32),
            0.0)
        xg = lax.dot_general(pg, xv, (((0,), (0,)), ((), ())),
                             preferred_element_type=jnp.float32)
        sg = lax.dot_general(pg, scale_mat, (((0,), (0,)), ((), ())),
                             preferred_element_type=jnp.float32)
        yg = (xg * sg[:, 0:1]) @ ew_ref[0]
        for e in range(1, E_LOCAL):
            yg += (xg * sg[:, e:e + 1]) @ ew_ref[e]

        for zk in range(N_Z):
            rs = zk * BLK
            route_b = ri_ref[pl.ds(rs, BLK), :]
            mine = lax.div(route_b, E_LOCAL) == d
            slots = tri_bd @ mine.astype(jnp.float32) - 1.0
            slots_i = slots.astype(jnp.int32)
            colidx = (lax.broadcasted_iota(jnp.int32, (BLK, 1), 0)
                      // ROWS) * CAP1 + slots_i
            ptb = jnp.where(jnp.logical_and(mine, slots_i < CAP1),
                            (colidx == iota_col4).astype(jnp.float32),
                            0.0)
            colg_b = colg[rs:rs + BLK]
            pgb = jnp.where(
                jnp.logical_and(mine, colg_b < CAP_ME),
                (colg_b == lax.broadcasted_iota(
                    jnp.int32, (BLK, CAP_ME), 1)).astype(jnp.float32),
                0.0)
            compose = lax.dot_general(ptb, pgb, (((0,), (0,)), ((), ())),
                                      preferred_element_type=jnp.float32)
            g = compose @ yg
            stage_ref[pl.ds(zk * PLANE * CAP1, PLANE * CAP1), :] = (
                g.astype(jnp.bfloat16))

            for wp in range(PLANE):
                c = 4 * zk + wp
                rdma = pltpu.make_async_remote_copy(
                    src_ref=stage_ref.at[pl.ds(c * CAP1, CAP1), :],
                    dst_ref=rp_ref.at[pl.ds(d * CAP1, CAP1), :],
                    send_sem=send_sems.at[c],
                    recv_sem=recv_sems.at[d],
                    device_id=(c,),
                    device_id_type=pl.DeviceIdType.MESH,
                )
                rdma.start()
                sends.append(rdma)

        route_m = ri_ref[pl.ds(d * ROWS, ROWS), :]
        own_m = lax.div(route_m, E_LOCAL)
        onehot = (own_m == lax.broadcasted_iota(jnp.int32, (ROWS, N_DEV), 1)
                  ).astype(jnp.float32)
        tri64 = (lax.broadcasted_iota(jnp.int32, (ROWS, ROWS), 1)
                 <= lax.broadcasted_iota(jnp.int32, (ROWS, ROWS), 0)
                 ).astype(jnp.float32)
        counts = tri64 @ onehot
        slot_m = (jnp.sum(onehot * counts, axis=-1, keepdims=True)
                  ).astype(jnp.int32) - 1
        colidx_m = own_m * CAP1 + slot_m
        ptc = jnp.where(
            slot_m < CAP1,
            (colidx_m == lax.broadcasted_iota(
                jnp.int32, (ROWS, N_DEV * CAP1), 1)).astype(jnp.float32),
            0.0)

        shared = x_ref[pl.ds(d * ROWS, ROWS), :] @ sw_ref[:, :]

        for j in range(N_DEV):
            recv = pltpu.make_async_remote_copy(
                src_ref=stage_ref.at[pl.ds(0, CAP1), :],
                dst_ref=rp_ref.at[pl.ds(j * CAP1, CAP1), :],
                send_sem=send_sems.at[j],
                recv_sem=recv_sems.at[j],
                device_id=(d,),
                device_id_type=pl.DeviceIdType.MESH,
            )
            recv.wait_recv()
        out_ref[:, :] = shared + ptc @ rp_ref[:, :].astype(jnp.float32)

        for rdma in sends:
            rdma.wait_send()

    return pl.pallas_call(
        body,
        out_shape=jax.ShapeDtypeStruct((ROWS, D_OUT), jnp.float32),
        in_specs=[pl.BlockSpec(memory_space=pltpu.VMEM)] * 5,
        out_specs=pl.BlockSpec(memory_space=pltpu.VMEM),
        scratch_shapes=[
            pltpu.VMEM((N_DEV * CAP1, D_OUT), jnp.bfloat16),
            pltpu.VMEM((N_DEV * CAP1, D_OUT), jnp.bfloat16),
            pltpu.SemaphoreType.DMA((N_DEV,)),
            pltpu.SemaphoreType.DMA((N_DEV,)),
        ],
        compiler_params=pltpu.CompilerParams(collective_id=0),
    )(x, router_W, route_idx, expert_W, shared_W)


# device time: 22467 ns/iter; 1.6832x vs baseline; 1.1357x over previous
import jax
import jax.numpy as jnp
from jax import lax
from jax.experimental import pallas as pl
from jax.experimental.pallas import tpu as pltpu

N_DEV = 16
PLANE = 4
N_Z = 4
N_TOK = 1024
D_IN = 512
D_OUT = 1024
E_LOCAL = 4
ROWS = N_TOK // N_DEV
BLK = PLANE * ROWS
CAP1 = 16
CAP_ME = 128


def kernel(x, router_W, route_idx, expert_W, shared_W):
    def body(x_ref, rw_ref, ri_ref, ew_ref, sw_ref, out_ref,
             stage_ref, rp_ref, send_sems, recv_sems):
        d = lax.axis_index("i")

        barrier = pltpu.get_barrier_semaphore()
        for j in range(N_DEV):
            pl.semaphore_signal(barrier, inc=1, device_id=(j,),
                                device_id_type=pl.DeviceIdType.MESH)

        xv = x_ref[:, :]
        scores = xv @ rw_ref[:, :]
        m = jnp.max(scores, axis=-1, keepdims=True)
        p = jnp.exp(scores - m)
        probs = p / jnp.sum(p, axis=-1, keepdims=True)
        route = ri_ref[:, :]
        eids = lax.broadcasted_iota(route.dtype, scores.shape, 1)
        coef = jnp.sum(jnp.where(eids == route, probs, 0.0),
                       axis=-1, keepdims=True)
        scale_mat = jnp.concatenate(
            [jnp.where(route == d * E_LOCAL + e, coef, 0.0)
             for e in range(E_LOCAL)], axis=-1)

        ri_b = lax.broadcasted_iota(jnp.int32, (BLK, BLK), 0)
        ci_b = lax.broadcasted_iota(jnp.int32, (BLK, BLK), 1)
        tri_bd = jnp.logical_and(ci_b <= ri_b,
                                 lax.div(ci_b, ROWS) == lax.div(ri_b, ROWS)
                                 ).astype(jnp.float32)
        tri_lo = (ci_b <= ri_b).astype(jnp.float32)
        iota_col4 = lax.broadcasted_iota(jnp.int32, (BLK, PLANE * CAP1), 1)

        sends = []

        mine_all = lax.div(route, E_LOCAL) == d
        minef = mine_all.astype(jnp.float32)
        colg_blocks = []
        off = jnp.float32(0.0)
        for b in range(N_Z):
            mb = minef[b * BLK:(b + 1) * BLK]
            slots_b = tri_lo @ mb - 1.0 + off
            colg_blocks.append(slots_b)
            off = off + jnp.sum(mb)
        colg = jnp.concatenate(colg_blocks, axis=0).astype(jnp.int32)
        pg = jnp.where(
            jnp.logical_and(mine_all, colg < CAP_ME),
            (colg == lax.broadcasted_iota(jnp.int32, (N_TOK, CAP_ME), 1)
             ).astype(jnp.float32),
            0.0)
        xg = lax.dot_general(pg, xv, (((0,), (0,)), ((), ())),
                             preferred_element_type=jnp.float32)
        sg = lax.dot_general(pg, scale_mat, (((0,), (0,)), ((), ())),
                             preferred_element_type=jnp.float32)
        yg = (xg * sg[:, 0:1]) @ ew_ref[0]
        for e in range(1, E_LOCAL):
            yg += (xg * sg[:, e:e + 1]) @ ew_ref[e]

        pl.semaphore_wait(barrier, N_DEV)

        for zk in range(N_Z):
            rs = zk * BLK
            route_b = ri_ref[pl.ds(rs, BLK), :]
            mine = lax.div(route_b, E_LOCAL) == d
            slots = tri_bd @ mine.astype(jnp.float32) - 1.0
            slots_i = slots.astype(jnp.int32)
            colidx = (lax.broadcasted_iota(jnp.int32, (BLK, 1), 0)
                      // ROWS) * CAP1 + slots_i
            ptb = jnp.where(jnp.logical_and(mine, slots_i < CAP1),
                            (colidx == iota_col4).astype(jnp.float32),
                            0.0)
            colg_b = colg[rs:rs + BLK]
            pgb = jnp.where(
                jnp.logical_and(mine, colg_b < CAP_ME),
                (colg_b == lax.broadcasted_iota(
                    jnp.int32, (BLK, CAP_ME), 1)).astype(jnp.float32),
                0.0)
            compose = lax.dot_general(ptb, pgb, (((0,), (0,)), ((), ())),
                                      preferred_element_type=jnp.float32)
            g = compose @ yg
            stage_ref[pl.ds(zk * PLANE * CAP1, PLANE * CAP1), :] = (
                g.astype(jnp.bfloat16))

            for wp in range(PLANE):
                c = 4 * zk + wp
                rdma = pltpu.make_async_remote_copy(
                    src_ref=stage_ref.at[pl.ds(c * CAP1, CAP1), :],
                    dst_ref=rp_ref.at[pl.ds(d * CAP1, CAP1), :],
                    send_sem=send_sems.at[c],
                    recv_sem=recv_sems.at[d],
                    device_id=(c,),
                    device_id_type=pl.DeviceIdType.MESH,
                )
                rdma.start()
                sends.append(rdma)

        route_m = ri_ref[pl.ds(d * ROWS, ROWS), :]
        own_m = lax.div(route_m, E_LOCAL)
        onehot = (own_m == lax.broadcasted_iota(jnp.int32, (ROWS, N_DEV), 1)
                  ).astype(jnp.float32)
        tri64 = (lax.broadcasted_iota(jnp.int32, (ROWS, ROWS), 1)
                 <= lax.broadcasted_iota(jnp.int32, (ROWS, ROWS), 0)
                 ).astype(jnp.float32)
        counts = tri64 @ onehot
        slot_m = (jnp.sum(onehot * counts, axis=-1, keepdims=True)
                  ).astype(jnp.int32) - 1
        colidx_m = own_m * CAP1 + slot_m
        ptc = jnp.where(
            slot_m < CAP1,
            (colidx_m == lax.broadcasted_iota(
                jnp.int32, (ROWS, N_DEV * CAP1), 1)).astype(jnp.float32),
            0.0)

        shared = x_ref[pl.ds(d * ROWS, ROWS), :] @ sw_ref[:, :]

        for j in range(N_DEV):
            recv = pltpu.make_async_remote_copy(
                src_ref=stage_ref.at[pl.ds(0, CAP1), :],
                dst_ref=rp_ref.at[pl.ds(j * CAP1, CAP1), :],
                send_sem=send_sems.at[j],
                recv_sem=recv_sems.at[j],
                device_id=(d,),
                device_id_type=pl.DeviceIdType.MESH,
            )
            recv.wait_recv()
        out_ref[:, :] = shared + ptc @ rp_ref[:, :].astype(jnp.float32)

        for rdma in sends:
            rdma.wait_send()

    return pl.pallas_call(
        body,
        out_shape=jax.ShapeDtypeStruct((ROWS, D_OUT), jnp.float32),
        in_specs=[pl.BlockSpec(memory_space=pltpu.VMEM)] * 5,
        out_specs=pl.BlockSpec(memory_space=pltpu.VMEM),
        scratch_shapes=[
            pltpu.VMEM((N_DEV * CAP1, D_OUT), jnp.bfloat16),
            pltpu.VMEM((N_DEV * CAP1, D_OUT), jnp.bfloat16),
            pltpu.SemaphoreType.DMA((N_DEV,)),
            pltpu.SemaphoreType.DMA((N_DEV,)),
        ],
        compiler_params=pltpu.CompilerParams(collective_id=0),
    )(x, router_W, route_idx, expert_W, shared_W)
